# baseline (speedup 1.0000x reference)
import functools

import jax
import jax.numpy as jnp
from jax import lax
from jax.experimental import pallas as pl
from jax.experimental.pallas import tpu as pltpu
from jax.experimental.pallas import tpu_sc as plsc

T, E, H = 32768, 1024, 1024
K = 64
NS = 16
CHUNK = T // NS
SUPER = 128
NSUP = CHUNK // SUPER
NEG = -3.0e38


_BT = 2048


def _alpha_body(excs_ref, exc_ref, out_ref):
    x = excs_ref[...]
    e = exc_ref[...]
    out_ref[...] = jnp.sum(x * e, axis=1)


def _compute_alpha(excs2d, exc2d):
    return pl.pallas_call(
        _alpha_body,
        grid=(T // _BT,),
        in_specs=[
            pl.BlockSpec((_BT, E), lambda i: (i, 0)),
            pl.BlockSpec((1, E), lambda i: (0, 0)),
        ],
        out_specs=pl.BlockSpec((_BT,), lambda i: (i,)),
        out_shape=jax.ShapeDtypeStruct((T,), jnp.float32),
    )(excs2d, exc2d)


_HC = 256


def _gru_body(wih_ref, whh_ref, bih_ref, bhh_ref, exc_ref, sc_ref, hf_ref,
              hb_ref, out_ref):
    i = pl.program_id(0)
    s = sc_ref[0, 0]
    m = jnp.where(s >= 0.5, jnp.float32(1.0), jnp.float32(0.0))
    e = exc_ref[...]
    x = jnp.concatenate([e * m, e * (1.0 - m)], axis=1)
    gx = jnp.sum(wih_ref[...] * x[None, :, :], axis=2)
    gh = jnp.sum(whh_ref[...] * hf_ref[...][None, :, :], axis=2)
    gx = gx + bih_ref[:, 0, pl.ds(i * _HC, _HC)]
    gh = gh + bhh_ref[:, 0, pl.ds(i * _HC, _HC)]
    r = jax.nn.sigmoid(gx[0] + gh[0])
    z = jax.nn.sigmoid(gx[1] + gh[1])
    n = jnp.tanh(gx[2] + r * gh[2])
    out_ref[...] = (1.0 - z) * n + z * hb_ref[0, :]


def _compute_gru(wih3, whh3, bih3, bhh3, exc2d, sc2d, hlast2d):
    return pl.pallas_call(
        _gru_body,
        grid=(H // _HC,),
        in_specs=[
            pl.BlockSpec((3, _HC, 2 * E), lambda i: (0, i, 0)),
            pl.BlockSpec((3, _HC, H), lambda i: (0, i, 0)),
            pl.BlockSpec((3, 1, H), lambda i: (0, 0, 0)),
            pl.BlockSpec((3, 1, H), lambda i: (0, 0, 0)),
            pl.BlockSpec((1, E), lambda i: (0, 0)),
            pl.BlockSpec((1, 1), lambda i: (0, 0)),
            pl.BlockSpec((1, H), lambda i: (0, 0)),
            pl.BlockSpec((1, _HC), lambda i: (0, i)),
        ],
        out_specs=pl.BlockSpec((_HC,), lambda i: (i,)),
        out_shape=jax.ShapeDtypeStruct((H,), jnp.float32),
    )(wih3, whh3, bih3, bhh3, exc2d, sc2d, hlast2d, hlast2d)


def _sc_body(alpha_hbm, hs_hbm, exc_hbm, wsc_hbm, bsc_hbm, out_hbm,
             a_v, lv_v, li_v, cand_sh, candi_sh, cv_v, ci_v, gv_v, gi_v,
             wv_v, w_sh, gi_sh, w_v, gi64_v, rows_v, wsct_v, exc64_v,
             wsch_v, part64_v, part_sh, pf_v, b_v, outv_v, sem):
    cid = lax.axis_index("c")
    sid = lax.axis_index("s")
    iota = lax.iota(jnp.int32, 16)

    @pl.when(cid == 0)
    def _core0():
        w = sid

        pltpu.sync_copy(alpha_hbm.at[pl.ds(w * CHUNK, CHUNK)], a_v)

        def _supermax(base):
            m = a_v[pl.ds(base, 16)]
            for j in range(1, SUPER // 16):
                m = jnp.maximum(m, a_v[pl.ds(base + j * 16, 16)])
            return jnp.max(m)

        M = jnp.full((16,), NEG, jnp.float32)
        for s in range(NSUP):
            M = jnp.where(iota == s, _supermax(s * SUPER), M)

        def _sel(k, M):
            gm = jnp.max(M)
            sc = jnp.min(jnp.where(M == gm, iota, 9999))
            pos = jnp.int32(9999)
            for j in range(SUPER // 16):
                vj = a_v[pl.ds(sc * SUPER + j * 16, 16)]
                pj = jnp.min(jnp.where(vj == gm, iota + j * 16,
                                       jnp.int32(9999)))
                pos = jnp.minimum(pos, pj)
            jj = pos // 16
            lane = pos - jj * 16
            off = sc * SUPER + jj * 16
            vv = a_v[pl.ds(off, 16)]
            a_v[pl.ds(off, 16)] = jnp.where(iota == lane, NEG, vv)
            nm = a_v[pl.ds(sc * SUPER, 16)]
            for j in range(1, SUPER // 16):
                nm = jnp.maximum(nm, a_v[pl.ds(sc * SUPER + j * 16, 16)])
            M2 = jnp.where(iota == sc, jnp.max(nm), M)
            blk = (k // 16) * 16
            l2 = k - blk
            lvb = lv_v[pl.ds(blk, 16)]
            lv_v[pl.ds(blk, 16)] = jnp.where(iota == l2, gm, lvb)
            lib = li_v[pl.ds(blk, 16)]
            li_v[pl.ds(blk, 16)] = jnp.where(
                iota == l2, w * CHUNK + sc * SUPER + pos, lib)
            return M2

        lax.fori_loop(0, K, _sel, M)

        pltpu.sync_copy(lv_v, cand_sh.at[w])
        pltpu.sync_copy(li_v, candi_sh.at[w])
        plsc.subcore_barrier()

        @pl.when(w == 0)
        def _merge():
            pltpu.sync_copy(cand_sh.at[pl.ds(0, NS)], cv_v)
            pltpu.sync_copy(candi_sh.at[pl.ds(0, NS)], ci_v)

            def _mbody(k, cur):
                heads = plsc.load_gather(cv_v, [iota, cur])
                gm = jnp.max(heads)
                l = jnp.min(jnp.where(heads == gm, iota, 9999))
                curl = jnp.sum(jnp.where(iota == l, cur, 0))
                giv = plsc.load_gather(
                    ci_v, [jnp.full((16,), l, jnp.int32),
                           jnp.full((16,), curl, jnp.int32)])
                blk = (k // 16) * 16
                l2 = k - blk
                gvb = gv_v[pl.ds(blk, 16)]
                gv_v[pl.ds(blk, 16)] = jnp.where(iota == l2, gm, gvb)
                gib = gi_v[pl.ds(blk, 16)]
                gi_v[pl.ds(blk, 16)] = jnp.where(iota == l2, giv, gib)
                return jnp.where(iota == l, cur + 1, cur)

            lax.fori_loop(0, K, _mbody, jnp.zeros((16,), jnp.int32))

            v0 = gv_v[pl.ds(0, 16)]
            v1 = gv_v[pl.ds(16, 16)]
            v2 = gv_v[pl.ds(32, 16)]
            v3 = gv_v[pl.ds(48, 16)]
            mx = jnp.max(jnp.maximum(jnp.maximum(v0, v1),
                                     jnp.maximum(v2, v3)))
            e0 = jnp.exp(v0 - mx)
            e1 = jnp.exp(v1 - mx)
            e2 = jnp.exp(v2 - mx)
            e3 = jnp.exp(v3 - mx)
            s_vec = jnp.full((16,), jnp.sum(e0 + e1 + e2 + e3), jnp.float32)
            inv = jnp.ones((16,), jnp.float32) / s_vec
            wv_v[pl.ds(0, 16)] = e0 * inv
            wv_v[pl.ds(16, 16)] = e1 * inv
            wv_v[pl.ds(32, 16)] = e2 * inv
            wv_v[pl.ds(48, 16)] = e3 * inv
            pltpu.sync_copy(wv_v, w_sh.at[0])
            pltpu.sync_copy(gi_v, gi_sh.at[0])

        plsc.subcore_barrier()

        pltpu.sync_copy(w_sh.at[0], w_v)
        pltpu.sync_copy(gi_sh.at[0], gi64_v)
        pltpu.sync_copy(wsc_hbm.at[pl.ds(E, H)], wsct_v)
        pltpu.sync_copy(exc_hbm.at[pl.ds(w * 64, 64)], exc64_v)
        pltpu.sync_copy(wsc_hbm.at[pl.ds(w * 64, 64)], wsch_v)

        j0 = w * 4
        blk = (j0 // 16) * 16
        lane0 = j0 - blk
        wb = w_v[pl.ds(blk, 16)]
        ib = gi64_v[pl.ds(blk, 16)]
        ws = []
        descs = []
        for q in range(4):
            wq = jnp.sum(jnp.where(iota == lane0 + q, wb, jnp.float32(0.0)))
            iq = jnp.sum(jnp.where(iota == lane0 + q, ib, 0))
            ws.append(wq)
            descs.append(pltpu.async_copy(hs_hbm.at[iq], rows_v.at[q], sem))
        for d in descs:
            d.wait()

        acc = jnp.zeros((16,), jnp.float32)
        for q in range(H // 16):
            sl = pl.ds(q * 16, 16)
            rowsum = (ws[0] * rows_v[0, sl] + ws[1] * rows_v[1, sl]
                      + ws[2] * rows_v[2, sl] + ws[3] * rows_v[3, sl])
            acc = acc + rowsum * wsct_v[sl]
        for q in range(4):
            sl = pl.ds(q * 16, 16)
            acc = acc + exc64_v[sl] * wsch_v[sl]
        p = jnp.sum(acc)
        pz = jnp.zeros((16,), jnp.float32)
        part64_v[pl.ds(0, 16)] = jnp.where(iota == 0, p, jnp.float32(0.0))
        part64_v[pl.ds(16, 16)] = pz
        part64_v[pl.ds(32, 16)] = pz
        part64_v[pl.ds(48, 16)] = pz
        pltpu.sync_copy(part64_v, part_sh.at[w])
        plsc.subcore_barrier()

        @pl.when(w == 0)
        def _final():
            pltpu.sync_copy(part_sh.at[pl.ds(0, NS)], pf_v)
            pltpu.sync_copy(bsc_hbm, b_v)
            tot = pf_v[0, pl.ds(0, 16)]
            for q in range(1, NS):
                tot = tot + pf_v[q, pl.ds(0, 16)]
            outv_v[...] = jnp.where(iota == 0, tot + b_v[...],
                                    jnp.float32(0.0))
            pltpu.sync_copy(outv_v, out_hbm)


def _compute_attention(alpha, hs2d, exc, wsc, bsc16):
    mesh = plsc.VectorSubcoreMesh(core_axis_name="c", subcore_axis_name="s")
    f32, i32 = jnp.float32, jnp.int32
    body = functools.partial(
        pl.kernel,
        out_type=jax.ShapeDtypeStruct((16,), f32),
        mesh=mesh,
        scratch_types=[
            pltpu.VMEM((CHUNK,), f32),
            pltpu.VMEM((K,), f32),
            pltpu.VMEM((K,), i32),
            pltpu.VMEM_SHARED((2 * NS, K), f32),
            pltpu.VMEM_SHARED((2 * NS, K), i32),
            pltpu.VMEM((NS, K), f32),
            pltpu.VMEM((NS, K), i32),
            pltpu.VMEM((K,), f32),
            pltpu.VMEM((K,), i32),
            pltpu.VMEM((K,), f32),
            pltpu.VMEM_SHARED((2, K), f32),
            pltpu.VMEM_SHARED((2, K), i32),
            pltpu.VMEM((K,), f32),
            pltpu.VMEM((K,), i32),
            pltpu.VMEM((4, H), f32),
            pltpu.VMEM((H,), f32),
            pltpu.VMEM((64,), f32),
            pltpu.VMEM((64,), f32),
            pltpu.VMEM((64,), f32),
            pltpu.VMEM_SHARED((2 * NS, 64), f32),
            pltpu.VMEM((NS, 64), f32),
            pltpu.VMEM((16,), f32),
            pltpu.VMEM((16,), f32),
            pltpu.SemaphoreType.DMA,
        ],
        compiler_params=pltpu.CompilerParams(needs_layout_passes=False),
    )(_sc_body)
    return body(alpha, hs2d, exc, wsc, bsc16)


def kernel(exc, score, excs, hs, W_ih, W_hh, b_ih, b_hh, W_score, b_score,
           attn_k):
    excs2d = excs.reshape(T, E)
    hs2d = hs.reshape(T, H)
    exc2d = exc.reshape(1, E)
    sc2d = score.reshape(1, 1)
    hlast2d = hs2d[T - 1:T, :]
    wih3 = W_ih.reshape(3, H, 2 * E)
    whh3 = W_hh.reshape(3, H, H)
    bih3 = b_ih.reshape(3, 1, H)
    bhh3 = b_hh.reshape(3, 1, H)
    wsc = W_score.reshape(2 * E)
    bsc16 = jnp.zeros((16,), jnp.float32).at[0].set(b_score[0])

    alpha = _compute_alpha(excs2d, exc2d)
    pred16 = _compute_attention(alpha, hs2d, exc2d.reshape(E), wsc, bsc16)
    h_new = _compute_gru(wih3, whh3, bih3, bhh3, exc2d, sc2d, hlast2d)

    pred = pred16[0:1].reshape(1, 1)
    return (pred, h_new.reshape(1, 1, H))


# speedup vs baseline: 1.5566x; 1.5566x over previous
import functools

import jax
import jax.numpy as jnp
from jax import lax
from jax.experimental import pallas as pl
from jax.experimental.pallas import tpu as pltpu
from jax.experimental.pallas import tpu_sc as plsc

T, E, H = 32768, 1024, 1024
K = 64
NS = 16
CHUNK = T // NS
SUPER = 128
NSUP = CHUNK // SUPER
NEG = -3.0e38


_BT = 2048


def _alpha_body(excs_ref, exc_ref, out_ref):
    x = excs_ref[...]
    e = exc_ref[...]
    out_ref[...] = jnp.sum(x * e[None], axis=(1, 2))


def _compute_alpha(excs3d, exc3d):
    return pl.pallas_call(
        _alpha_body,
        grid=(T // _BT,),
        in_specs=[
            pl.BlockSpec((_BT, 8, 128), lambda i: (i, 0, 0)),
            pl.BlockSpec((8, 128), lambda i: (0, 0)),
        ],
        out_specs=pl.BlockSpec((_BT,), lambda i: (i,)),
        out_shape=jax.ShapeDtypeStruct((T,), jnp.float32),
    )(excs3d, exc3d)


_HC = 256


def _gru_body(wih_ref, whh_ref, bih_ref, bhh_ref, exc_ref, sc_ref, hf_ref,
              hb_ref, out_ref):
    i = pl.program_id(0)
    s = sc_ref[0, 0]
    m = jnp.where(s >= 0.5, jnp.float32(1.0), jnp.float32(0.0))
    e = exc_ref[...]
    x = jnp.concatenate([e * m, e * (1.0 - m)], axis=1)
    gx = jnp.sum(wih_ref[...] * x[None, :, :], axis=2)
    gh = jnp.sum(whh_ref[...] * hf_ref[...][None, :, :], axis=2)
    gx = gx + bih_ref[:, 0, pl.ds(i * _HC, _HC)]
    gh = gh + bhh_ref[:, 0, pl.ds(i * _HC, _HC)]
    r = jax.nn.sigmoid(gx[0] + gh[0])
    z = jax.nn.sigmoid(gx[1] + gh[1])
    n = jnp.tanh(gx[2] + r * gh[2])
    out_ref[...] = (1.0 - z) * n + z * hb_ref[0, :]


def _compute_gru(wih3, whh3, bih3, bhh3, exc2d, sc2d, hlast2d):
    return pl.pallas_call(
        _gru_body,
        grid=(H // _HC,),
        in_specs=[
            pl.BlockSpec((3, _HC, 2 * E), lambda i: (0, i, 0)),
            pl.BlockSpec((3, _HC, H), lambda i: (0, i, 0)),
            pl.BlockSpec((3, 1, H), lambda i: (0, 0, 0)),
            pl.BlockSpec((3, 1, H), lambda i: (0, 0, 0)),
            pl.BlockSpec((1, E), lambda i: (0, 0)),
            pl.BlockSpec((1, 1), lambda i: (0, 0)),
            pl.BlockSpec((1, H), lambda i: (0, 0)),
            pl.BlockSpec((1, _HC), lambda i: (0, i)),
        ],
        out_specs=pl.BlockSpec((_HC,), lambda i: (i,)),
        out_shape=jax.ShapeDtypeStruct((H,), jnp.float32),
    )(wih3, whh3, bih3, bhh3, exc2d, sc2d, hlast2d, hlast2d)


def _sc_body(alpha_hbm, hs_hbm, exc_hbm, wsc_hbm, bsc_hbm, out_hbm,
             a_v, lv_v, li_v, cand_sh, candi_sh, cv_v, ci_v, gv_v, gi_v,
             wv_v, w_sh, gi_sh, w_v, gi64_v, rows_v, wsct_v, exc64_v,
             wsch_v, part64_v, part_sh, pf_v, b_v, outv_v, sem):
    cid = lax.axis_index("c")
    sid = lax.axis_index("s")
    iota = lax.iota(jnp.int32, 16)

    @pl.when(cid == 0)
    def _core0():
        w = sid

        pltpu.sync_copy(alpha_hbm.at[pl.ds(w * CHUNK, CHUNK)], a_v)

        def _supermax(base):
            m = a_v[pl.ds(base, 16)]
            for j in range(1, SUPER // 16):
                m = jnp.maximum(m, a_v[pl.ds(base + j * 16, 16)])
            return jnp.max(m)

        M = jnp.full((16,), NEG, jnp.float32)
        for s in range(NSUP):
            M = jnp.where(iota == s, _supermax(s * SUPER), M)

        def _sel(k, M):
            gm = jnp.max(M)
            sc = jnp.min(jnp.where(M == gm, iota, 9999))
            pos = jnp.int32(9999)
            for j in range(SUPER // 16):
                vj = a_v[pl.ds(sc * SUPER + j * 16, 16)]
                pj = jnp.min(jnp.where(vj == gm, iota + j * 16,
                                       jnp.int32(9999)))
                pos = jnp.minimum(pos, pj)
            jj = pos // 16
            lane = pos - jj * 16
            off = sc * SUPER + jj * 16
            vv = a_v[pl.ds(off, 16)]
            a_v[pl.ds(off, 16)] = jnp.where(iota == lane, NEG, vv)
            nm = a_v[pl.ds(sc * SUPER, 16)]
            for j in range(1, SUPER // 16):
                nm = jnp.maximum(nm, a_v[pl.ds(sc * SUPER + j * 16, 16)])
            M2 = jnp.where(iota == sc, jnp.max(nm), M)
            blk = (k // 16) * 16
            l2 = k - blk
            lvb = lv_v[pl.ds(blk, 16)]
            lv_v[pl.ds(blk, 16)] = jnp.where(iota == l2, gm, lvb)
            lib = li_v[pl.ds(blk, 16)]
            li_v[pl.ds(blk, 16)] = jnp.where(
                iota == l2, w * CHUNK + sc * SUPER + pos, lib)
            return M2

        lax.fori_loop(0, K, _sel, M)

        pltpu.sync_copy(lv_v, cand_sh.at[w])
        pltpu.sync_copy(li_v, candi_sh.at[w])
        plsc.subcore_barrier()

        @pl.when(w == 0)
        def _merge():
            pltpu.sync_copy(cand_sh.at[pl.ds(0, NS)], cv_v)
            pltpu.sync_copy(candi_sh.at[pl.ds(0, NS)], ci_v)

            def _mbody(k, cur):
                heads = plsc.load_gather(cv_v, [iota, cur])
                gm = jnp.max(heads)
                l = jnp.min(jnp.where(heads == gm, iota, 9999))
                curl = jnp.sum(jnp.where(iota == l, cur, 0))
                giv = plsc.load_gather(
                    ci_v, [jnp.full((16,), l, jnp.int32),
                           jnp.full((16,), curl, jnp.int32)])
                blk = (k // 16) * 16
                l2 = k - blk
                gvb = gv_v[pl.ds(blk, 16)]
                gv_v[pl.ds(blk, 16)] = jnp.where(iota == l2, gm, gvb)
                gib = gi_v[pl.ds(blk, 16)]
                gi_v[pl.ds(blk, 16)] = jnp.where(iota == l2, giv, gib)
                return jnp.where(iota == l, cur + 1, cur)

            lax.fori_loop(0, K, _mbody, jnp.zeros((16,), jnp.int32))

            v0 = gv_v[pl.ds(0, 16)]
            v1 = gv_v[pl.ds(16, 16)]
            v2 = gv_v[pl.ds(32, 16)]
            v3 = gv_v[pl.ds(48, 16)]
            mx = jnp.max(jnp.maximum(jnp.maximum(v0, v1),
                                     jnp.maximum(v2, v3)))
            e0 = jnp.exp(v0 - mx)
            e1 = jnp.exp(v1 - mx)
            e2 = jnp.exp(v2 - mx)
            e3 = jnp.exp(v3 - mx)
            s_vec = jnp.full((16,), jnp.sum(e0 + e1 + e2 + e3), jnp.float32)
            inv = jnp.ones((16,), jnp.float32) / s_vec
            wv_v[pl.ds(0, 16)] = e0 * inv
            wv_v[pl.ds(16, 16)] = e1 * inv
            wv_v[pl.ds(32, 16)] = e2 * inv
            wv_v[pl.ds(48, 16)] = e3 * inv
            pltpu.sync_copy(wv_v, w_sh.at[0])
            pltpu.sync_copy(gi_v, gi_sh.at[0])

        plsc.subcore_barrier()

        pltpu.sync_copy(w_sh.at[0], w_v)
        pltpu.sync_copy(gi_sh.at[0], gi64_v)
        pltpu.sync_copy(wsc_hbm.at[pl.ds(E, H)], wsct_v)
        pltpu.sync_copy(exc_hbm.at[pl.ds(w * 64, 64)], exc64_v)
        pltpu.sync_copy(wsc_hbm.at[pl.ds(w * 64, 64)], wsch_v)

        j0 = w * 4
        blk = (j0 // 16) * 16
        lane0 = j0 - blk
        wb = w_v[pl.ds(blk, 16)]
        ib = gi64_v[pl.ds(blk, 16)]
        ws = []
        descs = []
        for q in range(4):
            wq = jnp.sum(jnp.where(iota == lane0 + q, wb, jnp.float32(0.0)))
            iq = jnp.sum(jnp.where(iota == lane0 + q, ib, 0))
            ws.append(wq)
            descs.append(pltpu.async_copy(
                hs_hbm.at[pl.ds(iq * H, H)], rows_v.at[q], sem))
        for d in descs:
            d.wait()

        acc = jnp.zeros((16,), jnp.float32)
        for q in range(H // 16):
            sl = pl.ds(q * 16, 16)
            rowsum = (ws[0] * rows_v[0, sl] + ws[1] * rows_v[1, sl]
                      + ws[2] * rows_v[2, sl] + ws[3] * rows_v[3, sl])
            acc = acc + rowsum * wsct_v[sl]
        for q in range(4):
            sl = pl.ds(q * 16, 16)
            acc = acc + exc64_v[sl] * wsch_v[sl]
        p = jnp.sum(acc)
        pz = jnp.zeros((16,), jnp.float32)
        part64_v[pl.ds(0, 16)] = jnp.where(iota == 0, p, jnp.float32(0.0))
        part64_v[pl.ds(16, 16)] = pz
        part64_v[pl.ds(32, 16)] = pz
        part64_v[pl.ds(48, 16)] = pz
        pltpu.sync_copy(part64_v, part_sh.at[w])
        plsc.subcore_barrier()

        @pl.when(w == 0)
        def _final():
            pltpu.sync_copy(part_sh.at[pl.ds(0, NS)], pf_v)
            pltpu.sync_copy(bsc_hbm, b_v)
            tot = pf_v[0, pl.ds(0, 16)]
            for q in range(1, NS):
                tot = tot + pf_v[q, pl.ds(0, 16)]
            outv_v[...] = jnp.where(iota == 0, tot + b_v[...],
                                    jnp.float32(0.0))
            pltpu.sync_copy(outv_v, out_hbm)


def _compute_attention(alpha, hs2d, exc, wsc, bsc16):
    mesh = plsc.VectorSubcoreMesh(core_axis_name="c", subcore_axis_name="s")
    f32, i32 = jnp.float32, jnp.int32
    body = functools.partial(
        pl.kernel,
        out_type=jax.ShapeDtypeStruct((16,), f32),
        mesh=mesh,
        scratch_types=[
            pltpu.VMEM((CHUNK,), f32),
            pltpu.VMEM((K,), f32),
            pltpu.VMEM((K,), i32),
            pltpu.VMEM_SHARED((2 * NS, K), f32),
            pltpu.VMEM_SHARED((2 * NS, K), i32),
            pltpu.VMEM((NS, K), f32),
            pltpu.VMEM((NS, K), i32),
            pltpu.VMEM((K,), f32),
            pltpu.VMEM((K,), i32),
            pltpu.VMEM((K,), f32),
            pltpu.VMEM_SHARED((2, K), f32),
            pltpu.VMEM_SHARED((2, K), i32),
            pltpu.VMEM((K,), f32),
            pltpu.VMEM((K,), i32),
            pltpu.VMEM((4, H), f32),
            pltpu.VMEM((H,), f32),
            pltpu.VMEM((64,), f32),
            pltpu.VMEM((64,), f32),
            pltpu.VMEM((64,), f32),
            pltpu.VMEM_SHARED((2 * NS, 64), f32),
            pltpu.VMEM((NS, 64), f32),
            pltpu.VMEM((16,), f32),
            pltpu.VMEM((16,), f32),
            pltpu.SemaphoreType.DMA,
        ],
        compiler_params=pltpu.CompilerParams(needs_layout_passes=False),
    )(_sc_body)
    return body(alpha, hs2d, exc, wsc, bsc16)


def kernel(exc, score, excs, hs, W_ih, W_hh, b_ih, b_hh, W_score, b_score,
           attn_k):
    excs3d = excs.reshape(T, 8, 128)
    hs1d = hs.reshape(T * H)
    hs2d = hs.reshape(T, H)
    exc2d = exc.reshape(1, E)
    exc3d = exc.reshape(8, 128)
    sc2d = score.reshape(1, 1)
    hlast2d = hs2d[T - 1:T, :]
    wih3 = W_ih.reshape(3, H, 2 * E)
    whh3 = W_hh.reshape(3, H, H)
    bih3 = b_ih.reshape(3, 1, H)
    bhh3 = b_hh.reshape(3, 1, H)
    wsc = W_score.reshape(2 * E)
    bsc16 = jnp.zeros((16,), jnp.float32).at[0].set(b_score[0])

    alpha = _compute_alpha(excs3d, exc3d)
    pred16 = _compute_attention(alpha, hs1d, exc, wsc, bsc16)
    h_new = _compute_gru(wih3, whh3, bih3, bhh3, exc2d, sc2d, hlast2d)

    pred = pred16[0:1].reshape(1, 1)
    return (pred, h_new.reshape(1, 1, H))


# speedup vs baseline: 3.0551x; 1.9627x over previous
import functools

import jax
import jax.numpy as jnp
from jax import lax
from jax.experimental import pallas as pl
from jax.experimental.pallas import tpu as pltpu
from jax.experimental.pallas import tpu_sc as plsc

T, E, H = 32768, 1024, 1024
K = 64
NS = 16
CHUNK = T // NS
SUPER = 128
NSUP = CHUNK // SUPER
NEG = -3.0e38


_BT = 2048


def _alpha_body(excs_ref, exc_ref, out_ref):
    x = excs_ref[...].reshape(_BT, 8, 128)
    e = exc_ref[...]
    out_ref[...] = jnp.sum(x * e[None], axis=(1, 2))


def _compute_alpha(excs1d, exc3d):
    return pl.pallas_call(
        _alpha_body,
        grid=(T // _BT,),
        in_specs=[
            pl.BlockSpec((_BT * E,), lambda i: (i,)),
            pl.BlockSpec((8, 128), lambda i: (0, 0)),
        ],
        out_specs=pl.BlockSpec((_BT,), lambda i: (i,)),
        out_shape=jax.ShapeDtypeStruct((T,), jnp.float32),
    )(excs1d, exc3d)


_HC = 256


def _gru_body(wih_ref, whh_ref, bih_ref, bhh_ref, exc_ref, sc_ref, hf_ref,
              hb_ref, out_ref):
    i = pl.program_id(0)
    s = sc_ref[0, 0]
    m = jnp.where(s >= 0.5, jnp.float32(1.0), jnp.float32(0.0))
    e = exc_ref[...]
    x = jnp.concatenate([e * m, e * (1.0 - m)], axis=1)
    gx = jnp.sum(wih_ref[...] * x[None, :, :], axis=2)
    gh = jnp.sum(whh_ref[...] * hf_ref[...][None, :, :], axis=2)
    gx = gx + bih_ref[:, 0, pl.ds(i * _HC, _HC)]
    gh = gh + bhh_ref[:, 0, pl.ds(i * _HC, _HC)]
    r = jax.nn.sigmoid(gx[0] + gh[0])
    z = jax.nn.sigmoid(gx[1] + gh[1])
    n = jnp.tanh(gx[2] + r * gh[2])
    out_ref[...] = (1.0 - z) * n + z * hb_ref[0, :]


def _compute_gru(wih3, whh3, bih3, bhh3, exc2d, sc2d, hlast2d):
    return pl.pallas_call(
        _gru_body,
        grid=(H // _HC,),
        in_specs=[
            pl.BlockSpec((3, _HC, 2 * E), lambda i: (0, i, 0)),
            pl.BlockSpec((3, _HC, H), lambda i: (0, i, 0)),
            pl.BlockSpec((3, 1, H), lambda i: (0, 0, 0)),
            pl.BlockSpec((3, 1, H), lambda i: (0, 0, 0)),
            pl.BlockSpec((1, E), lambda i: (0, 0)),
            pl.BlockSpec((1, 1), lambda i: (0, 0)),
            pl.BlockSpec((1, H), lambda i: (0, 0)),
            pl.BlockSpec((1, _HC), lambda i: (0, i)),
        ],
        out_specs=pl.BlockSpec((_HC,), lambda i: (i,)),
        out_shape=jax.ShapeDtypeStruct((H,), jnp.float32),
    )(wih3, whh3, bih3, bhh3, exc2d, sc2d, hlast2d, hlast2d)


def _sc_body(alpha_hbm, hs_hbm, exc_hbm, wsc_hbm, bsc_hbm, out_hbm,
             a_v, lv_v, li_v, cand_sh, candi_sh, cv_v, ci_v, gv_v, gi_v,
             wv_v, w_sh, gi_sh, w_v, gi64_v, rows_v, wsct_v, exc64_v,
             wsch_v, part64_v, part_sh, pf_v, b_v, outv_v, sem):
    cid = lax.axis_index("c")
    sid = lax.axis_index("s")
    iota = lax.iota(jnp.int32, 16)

    @pl.when(cid == 0)
    def _core0():
        w = sid

        pltpu.sync_copy(alpha_hbm.at[pl.ds(w * CHUNK, CHUNK)], a_v)

        def _supermax(base):
            m = a_v[pl.ds(base, 16)]
            for j in range(1, SUPER // 16):
                m = jnp.maximum(m, a_v[pl.ds(base + j * 16, 16)])
            return jnp.max(m)

        M = jnp.full((16,), NEG, jnp.float32)
        for s in range(NSUP):
            M = jnp.where(iota == s, _supermax(s * SUPER), M)

        def _sel(k, M):
            gm = jnp.max(M)
            sc = jnp.min(jnp.where(M == gm, iota, 9999))
            pos = jnp.int32(9999)
            for j in range(SUPER // 16):
                vj = a_v[pl.ds(sc * SUPER + j * 16, 16)]
                pj = jnp.min(jnp.where(vj == gm, iota + j * 16,
                                       jnp.int32(9999)))
                pos = jnp.minimum(pos, pj)
            jj = pos // 16
            lane = pos - jj * 16
            off = sc * SUPER + jj * 16
            vv = a_v[pl.ds(off, 16)]
            a_v[pl.ds(off, 16)] = jnp.where(iota == lane, NEG, vv)
            nm = a_v[pl.ds(sc * SUPER, 16)]
            for j in range(1, SUPER // 16):
                nm = jnp.maximum(nm, a_v[pl.ds(sc * SUPER + j * 16, 16)])
            M2 = jnp.where(iota == sc, jnp.max(nm), M)
            blk = (k // 16) * 16
            l2 = k - blk
            lvb = lv_v[pl.ds(blk, 16)]
            lv_v[pl.ds(blk, 16)] = jnp.where(iota == l2, gm, lvb)
            lib = li_v[pl.ds(blk, 16)]
            li_v[pl.ds(blk, 16)] = jnp.where(
                iota == l2, w * CHUNK + sc * SUPER + pos, lib)
            return M2

        lax.fori_loop(0, K, _sel, M)

        pltpu.sync_copy(lv_v, cand_sh.at[w])
        pltpu.sync_copy(li_v, candi_sh.at[w])
        plsc.subcore_barrier()

        @pl.when(w == 0)
        def _merge():
            pltpu.sync_copy(cand_sh.at[pl.ds(0, NS)], cv_v)
            pltpu.sync_copy(candi_sh.at[pl.ds(0, NS)], ci_v)

            def _mbody(k, cur):
                heads = plsc.load_gather(cv_v, [iota, cur])
                gm = jnp.max(heads)
                l = jnp.min(jnp.where(heads == gm, iota, 9999))
                curl = jnp.sum(jnp.where(iota == l, cur, 0))
                giv = plsc.load_gather(
                    ci_v, [jnp.full((16,), l, jnp.int32),
                           jnp.full((16,), curl, jnp.int32)])
                blk = (k // 16) * 16
                l2 = k - blk
                gvb = gv_v[pl.ds(blk, 16)]
                gv_v[pl.ds(blk, 16)] = jnp.where(iota == l2, gm, gvb)
                gib = gi_v[pl.ds(blk, 16)]
                gi_v[pl.ds(blk, 16)] = jnp.where(iota == l2, giv, gib)
                return jnp.where(iota == l, cur + 1, cur)

            lax.fori_loop(0, K, _mbody, jnp.zeros((16,), jnp.int32))

            v0 = gv_v[pl.ds(0, 16)]
            v1 = gv_v[pl.ds(16, 16)]
            v2 = gv_v[pl.ds(32, 16)]
            v3 = gv_v[pl.ds(48, 16)]
            mx = jnp.max(jnp.maximum(jnp.maximum(v0, v1),
                                     jnp.maximum(v2, v3)))
            e0 = jnp.exp(v0 - mx)
            e1 = jnp.exp(v1 - mx)
            e2 = jnp.exp(v2 - mx)
            e3 = jnp.exp(v3 - mx)
            s_vec = jnp.full((16,), jnp.sum(e0 + e1 + e2 + e3), jnp.float32)
            inv = jnp.ones((16,), jnp.float32) / s_vec
            wv_v[pl.ds(0, 16)] = e0 * inv
            wv_v[pl.ds(16, 16)] = e1 * inv
            wv_v[pl.ds(32, 16)] = e2 * inv
            wv_v[pl.ds(48, 16)] = e3 * inv
            pltpu.sync_copy(wv_v, w_sh.at[0])
            pltpu.sync_copy(gi_v, gi_sh.at[0])

        plsc.subcore_barrier()

        pltpu.sync_copy(w_sh.at[0], w_v)
        pltpu.sync_copy(gi_sh.at[0], gi64_v)
        pltpu.sync_copy(wsc_hbm.at[pl.ds(E, H)], wsct_v)
        pltpu.sync_copy(exc_hbm.at[pl.ds(w * 64, 64)], exc64_v)
        pltpu.sync_copy(wsc_hbm.at[pl.ds(w * 64, 64)], wsch_v)

        j0 = w * 4
        blk = (j0 // 16) * 16
        lane0 = j0 - blk
        wb = w_v[pl.ds(blk, 16)]
        ib = gi64_v[pl.ds(blk, 16)]
        ws = []
        descs = []
        for q in range(4):
            wq = jnp.sum(jnp.where(iota == lane0 + q, wb, jnp.float32(0.0)))
            iq = jnp.sum(jnp.where(iota == lane0 + q, ib, 0))
            ws.append(wq)
            descs.append(pltpu.async_copy(
                hs_hbm.at[pl.ds(iq * H, H)], rows_v.at[q], sem))
        for d in descs:
            d.wait()

        acc = jnp.zeros((16,), jnp.float32)
        for q in range(H // 16):
            sl = pl.ds(q * 16, 16)
            rowsum = (ws[0] * rows_v[0, sl] + ws[1] * rows_v[1, sl]
                      + ws[2] * rows_v[2, sl] + ws[3] * rows_v[3, sl])
            acc = acc + rowsum * wsct_v[sl]
        for q in range(4):
            sl = pl.ds(q * 16, 16)
            acc = acc + exc64_v[sl] * wsch_v[sl]
        p = jnp.sum(acc)
        pz = jnp.zeros((16,), jnp.float32)
        part64_v[pl.ds(0, 16)] = jnp.where(iota == 0, p, jnp.float32(0.0))
        part64_v[pl.ds(16, 16)] = pz
        part64_v[pl.ds(32, 16)] = pz
        part64_v[pl.ds(48, 16)] = pz
        pltpu.sync_copy(part64_v, part_sh.at[w])
        plsc.subcore_barrier()

        @pl.when(w == 0)
        def _final():
            pltpu.sync_copy(part_sh.at[pl.ds(0, NS)], pf_v)
            pltpu.sync_copy(bsc_hbm, b_v)
            tot = pf_v[0, pl.ds(0, 16)]
            for q in range(1, NS):
                tot = tot + pf_v[q, pl.ds(0, 16)]
            outv_v[...] = jnp.where(iota == 0, tot + b_v[...],
                                    jnp.float32(0.0))
            pltpu.sync_copy(outv_v, out_hbm)


def _compute_attention(alpha, hs2d, exc, wsc, bsc16):
    mesh = plsc.VectorSubcoreMesh(core_axis_name="c", subcore_axis_name="s")
    f32, i32 = jnp.float32, jnp.int32
    body = functools.partial(
        pl.kernel,
        out_type=jax.ShapeDtypeStruct((16,), f32),
        mesh=mesh,
        scratch_types=[
            pltpu.VMEM((CHUNK,), f32),
            pltpu.VMEM((K,), f32),
            pltpu.VMEM((K,), i32),
            pltpu.VMEM_SHARED((2 * NS, K), f32),
            pltpu.VMEM_SHARED((2 * NS, K), i32),
            pltpu.VMEM((NS, K), f32),
            pltpu.VMEM((NS, K), i32),
            pltpu.VMEM((K,), f32),
            pltpu.VMEM((K,), i32),
            pltpu.VMEM((K,), f32),
            pltpu.VMEM_SHARED((2, K), f32),
            pltpu.VMEM_SHARED((2, K), i32),
            pltpu.VMEM((K,), f32),
            pltpu.VMEM((K,), i32),
            pltpu.VMEM((4, H), f32),
            pltpu.VMEM((H,), f32),
            pltpu.VMEM((64,), f32),
            pltpu.VMEM((64,), f32),
            pltpu.VMEM((64,), f32),
            pltpu.VMEM_SHARED((2 * NS, 64), f32),
            pltpu.VMEM((NS, 64), f32),
            pltpu.VMEM((16,), f32),
            pltpu.VMEM((16,), f32),
            pltpu.SemaphoreType.DMA,
        ],
        compiler_params=pltpu.CompilerParams(needs_layout_passes=False),
    )(_sc_body)
    return body(alpha, hs2d, exc, wsc, bsc16)


def kernel(exc, score, excs, hs, W_ih, W_hh, b_ih, b_hh, W_score, b_score,
           attn_k):
    excs1d = excs.reshape(T * E)
    hs1d = hs.reshape(T * H)
    exc2d = exc.reshape(1, E)
    exc3d = exc.reshape(8, 128)
    sc2d = score.reshape(1, 1)
    hlast2d = hs1d[(T - 1) * H:].reshape(1, H)
    wih3 = W_ih.reshape(3, H, 2 * E)
    whh3 = W_hh.reshape(3, H, H)
    bih3 = b_ih.reshape(3, 1, H)
    bhh3 = b_hh.reshape(3, 1, H)
    wsc = W_score.reshape(2 * E)
    bsc16 = jnp.zeros((16,), jnp.float32).at[0].set(b_score[0])

    alpha = _compute_alpha(excs1d, exc3d)
    pred16 = _compute_attention(alpha, hs1d, exc, wsc, bsc16)
    h_new = _compute_gru(wih3, whh3, bih3, bhh3, exc2d, sc2d, hlast2d)

    pred = pred16[0:1].reshape(1, 1)
    return (pred, h_new.reshape(1, 1, H))


# speedup vs baseline: 3.1267x; 1.0234x over previous
import functools

import jax
import jax.numpy as jnp
from jax import lax
from jax.experimental import pallas as pl
from jax.experimental.pallas import tpu as pltpu
from jax.experimental.pallas import tpu_sc as plsc

T, E, H = 32768, 1024, 1024
K = 64
NS = 16
CHUNK = T // NS
SUPER = 128
NSUP = CHUNK // SUPER
NEG = -3.0e38


_BT = 4096


def _alpha_body(excs_ref, exc_ref, out_ref):
    x = excs_ref[...].reshape(_BT, 8, 128)
    e = exc_ref[...]
    out_ref[...] = jnp.sum(x * e[None], axis=(1, 2))


def _compute_alpha(excs1d, exc3d):
    return pl.pallas_call(
        _alpha_body,
        grid=(T // _BT,),
        in_specs=[
            pl.BlockSpec((_BT * E,), lambda i: (i,)),
            pl.BlockSpec((8, 128), lambda i: (0, 0)),
        ],
        out_specs=pl.BlockSpec((_BT,), lambda i: (i,)),
        out_shape=jax.ShapeDtypeStruct((T,), jnp.float32),
    )(excs1d, exc3d)


_HC = 256


def _gru_body(wih_ref, whh_ref, bih_ref, bhh_ref, exc_ref, sc_ref, hf_ref,
              hb_ref, out_ref):
    i = pl.program_id(0)
    s = sc_ref[0, 0]
    m = jnp.where(s >= 0.5, jnp.float32(1.0), jnp.float32(0.0))
    e = exc_ref[...]
    x = jnp.concatenate([e * m, e * (1.0 - m)], axis=1)
    gx = jnp.sum(wih_ref[...] * x[None, :, :], axis=2)
    gh = jnp.sum(whh_ref[...] * hf_ref[...][None, :, :], axis=2)
    gx = gx + bih_ref[:, 0, pl.ds(i * _HC, _HC)]
    gh = gh + bhh_ref[:, 0, pl.ds(i * _HC, _HC)]
    r = jax.nn.sigmoid(gx[0] + gh[0])
    z = jax.nn.sigmoid(gx[1] + gh[1])
    n = jnp.tanh(gx[2] + r * gh[2])
    out_ref[...] = (1.0 - z) * n + z * hb_ref[0, :]


def _compute_gru(wih3, whh3, bih3, bhh3, exc2d, sc2d, hlast2d):
    return pl.pallas_call(
        _gru_body,
        grid=(H // _HC,),
        in_specs=[
            pl.BlockSpec((3, _HC, 2 * E), lambda i: (0, i, 0)),
            pl.BlockSpec((3, _HC, H), lambda i: (0, i, 0)),
            pl.BlockSpec((3, 1, H), lambda i: (0, 0, 0)),
            pl.BlockSpec((3, 1, H), lambda i: (0, 0, 0)),
            pl.BlockSpec((1, E), lambda i: (0, 0)),
            pl.BlockSpec((1, 1), lambda i: (0, 0)),
            pl.BlockSpec((1, H), lambda i: (0, 0)),
            pl.BlockSpec((1, _HC), lambda i: (0, i)),
        ],
        out_specs=pl.BlockSpec((_HC,), lambda i: (i,)),
        out_shape=jax.ShapeDtypeStruct((H,), jnp.float32),
    )(wih3, whh3, bih3, bhh3, exc2d, sc2d, hlast2d, hlast2d)


def _sc_body(alpha_hbm, hs_hbm, exc_hbm, wsc_hbm, bsc_hbm, out_hbm,
             a_v, lv_v, li_v, cand_sh, candi_sh, cv_v, ci_v, gv_v, gi_v,
             wv_v, w_sh, gi_sh, w_v, gi64_v, rows_v, wsct_v, exc64_v,
             wsch_v, part64_v, part_sh, pf_v, b_v, outv_v, sem):
    cid = lax.axis_index("c")
    sid = lax.axis_index("s")
    iota = lax.iota(jnp.int32, 16)

    @pl.when(cid == 0)
    def _core0():
        w = sid

        pltpu.sync_copy(alpha_hbm.at[pl.ds(w * CHUNK, CHUNK)], a_v)

        def _supermax(base):
            m = a_v[pl.ds(base, 16)]
            for j in range(1, SUPER // 16):
                m = jnp.maximum(m, a_v[pl.ds(base + j * 16, 16)])
            return jnp.max(m)

        M = jnp.full((16,), NEG, jnp.float32)
        for s in range(NSUP):
            M = jnp.where(iota == s, _supermax(s * SUPER), M)

        def _sel(k, M):
            gm = jnp.max(M)
            sc = jnp.min(jnp.where(M == gm, iota, 9999))
            pos = jnp.int32(9999)
            for j in range(SUPER // 16):
                vj = a_v[pl.ds(sc * SUPER + j * 16, 16)]
                pj = jnp.min(jnp.where(vj == gm, iota + j * 16,
                                       jnp.int32(9999)))
                pos = jnp.minimum(pos, pj)
            jj = pos // 16
            lane = pos - jj * 16
            off = sc * SUPER + jj * 16
            vv = a_v[pl.ds(off, 16)]
            a_v[pl.ds(off, 16)] = jnp.where(iota == lane, NEG, vv)
            nm = a_v[pl.ds(sc * SUPER, 16)]
            for j in range(1, SUPER // 16):
                nm = jnp.maximum(nm, a_v[pl.ds(sc * SUPER + j * 16, 16)])
            M2 = jnp.where(iota == sc, jnp.max(nm), M)
            blk = (k // 16) * 16
            l2 = k - blk
            lvb = lv_v[pl.ds(blk, 16)]
            lv_v[pl.ds(blk, 16)] = jnp.where(iota == l2, gm, lvb)
            lib = li_v[pl.ds(blk, 16)]
            li_v[pl.ds(blk, 16)] = jnp.where(
                iota == l2, w * CHUNK + sc * SUPER + pos, lib)
            return M2

        lax.fori_loop(0, K, _sel, M)

        pltpu.sync_copy(lv_v, cand_sh.at[w])
        pltpu.sync_copy(li_v, candi_sh.at[w])
        plsc.subcore_barrier()

        @pl.when(w == 0)
        def _merge():
            pltpu.sync_copy(cand_sh.at[pl.ds(0, NS)], cv_v)
            pltpu.sync_copy(candi_sh.at[pl.ds(0, NS)], ci_v)

            def _mbody(k, cur):
                heads = plsc.load_gather(cv_v, [iota, cur])
                gm = jnp.max(heads)
                l = jnp.min(jnp.where(heads == gm, iota, 9999))
                curl = jnp.sum(jnp.where(iota == l, cur, 0))
                giv = plsc.load_gather(
                    ci_v, [jnp.full((16,), l, jnp.int32),
                           jnp.full((16,), curl, jnp.int32)])
                blk = (k // 16) * 16
                l2 = k - blk
                gvb = gv_v[pl.ds(blk, 16)]
                gv_v[pl.ds(blk, 16)] = jnp.where(iota == l2, gm, gvb)
                gib = gi_v[pl.ds(blk, 16)]
                gi_v[pl.ds(blk, 16)] = jnp.where(iota == l2, giv, gib)
                return jnp.where(iota == l, cur + 1, cur)

            lax.fori_loop(0, K, _mbody, jnp.zeros((16,), jnp.int32))

            v0 = gv_v[pl.ds(0, 16)]
            v1 = gv_v[pl.ds(16, 16)]
            v2 = gv_v[pl.ds(32, 16)]
            v3 = gv_v[pl.ds(48, 16)]
            mx = jnp.max(jnp.maximum(jnp.maximum(v0, v1),
                                     jnp.maximum(v2, v3)))
            e0 = jnp.exp(v0 - mx)
            e1 = jnp.exp(v1 - mx)
            e2 = jnp.exp(v2 - mx)
            e3 = jnp.exp(v3 - mx)
            s_vec = jnp.full((16,), jnp.sum(e0 + e1 + e2 + e3), jnp.float32)
            inv = jnp.ones((16,), jnp.float32) / s_vec
            wv_v[pl.ds(0, 16)] = e0 * inv
            wv_v[pl.ds(16, 16)] = e1 * inv
            wv_v[pl.ds(32, 16)] = e2 * inv
            wv_v[pl.ds(48, 16)] = e3 * inv
            pltpu.sync_copy(wv_v, w_sh.at[0])
            pltpu.sync_copy(gi_v, gi_sh.at[0])

        plsc.subcore_barrier()

        pltpu.sync_copy(w_sh.at[0], w_v)
        pltpu.sync_copy(gi_sh.at[0], gi64_v)
        pltpu.sync_copy(wsc_hbm.at[pl.ds(E, H)], wsct_v)
        pltpu.sync_copy(exc_hbm.at[pl.ds(w * 64, 64)], exc64_v)
        pltpu.sync_copy(wsc_hbm.at[pl.ds(w * 64, 64)], wsch_v)

        j0 = w * 4
        blk = (j0 // 16) * 16
        lane0 = j0 - blk
        wb = w_v[pl.ds(blk, 16)]
        ib = gi64_v[pl.ds(blk, 16)]
        ws = []
        descs = []
        for q in range(4):
            wq = jnp.sum(jnp.where(iota == lane0 + q, wb, jnp.float32(0.0)))
            iq = jnp.sum(jnp.where(iota == lane0 + q, ib, 0))
            ws.append(wq)
            descs.append(pltpu.async_copy(
                hs_hbm.at[pl.ds(iq * H, H)], rows_v.at[q], sem))
        for d in descs:
            d.wait()

        acc = jnp.zeros((16,), jnp.float32)
        for q in range(H // 16):
            sl = pl.ds(q * 16, 16)
            rowsum = (ws[0] * rows_v[0, sl] + ws[1] * rows_v[1, sl]
                      + ws[2] * rows_v[2, sl] + ws[3] * rows_v[3, sl])
            acc = acc + rowsum * wsct_v[sl]
        for q in range(4):
            sl = pl.ds(q * 16, 16)
            acc = acc + exc64_v[sl] * wsch_v[sl]
        p = jnp.sum(acc)
        pz = jnp.zeros((16,), jnp.float32)
        part64_v[pl.ds(0, 16)] = jnp.where(iota == 0, p, jnp.float32(0.0))
        part64_v[pl.ds(16, 16)] = pz
        part64_v[pl.ds(32, 16)] = pz
        part64_v[pl.ds(48, 16)] = pz
        pltpu.sync_copy(part64_v, part_sh.at[w])
        plsc.subcore_barrier()

        @pl.when(w == 0)
        def _final():
            pltpu.sync_copy(part_sh.at[pl.ds(0, NS)], pf_v)
            pltpu.sync_copy(bsc_hbm, b_v)
            tot = pf_v[0, pl.ds(0, 16)]
            for q in range(1, NS):
                tot = tot + pf_v[q, pl.ds(0, 16)]
            outv_v[...] = jnp.where(iota == 0, tot + b_v[...],
                                    jnp.float32(0.0))
            pltpu.sync_copy(outv_v, out_hbm)


def _compute_attention(alpha, hs2d, exc, wsc, bsc16):
    mesh = plsc.VectorSubcoreMesh(core_axis_name="c", subcore_axis_name="s")
    f32, i32 = jnp.float32, jnp.int32
    body = functools.partial(
        pl.kernel,
        out_type=jax.ShapeDtypeStruct((16,), f32),
        mesh=mesh,
        scratch_types=[
            pltpu.VMEM((CHUNK,), f32),
            pltpu.VMEM((K,), f32),
            pltpu.VMEM((K,), i32),
            pltpu.VMEM_SHARED((2 * NS, K), f32),
            pltpu.VMEM_SHARED((2 * NS, K), i32),
            pltpu.VMEM((NS, K), f32),
            pltpu.VMEM((NS, K), i32),
            pltpu.VMEM((K,), f32),
            pltpu.VMEM((K,), i32),
            pltpu.VMEM((K,), f32),
            pltpu.VMEM_SHARED((2, K), f32),
            pltpu.VMEM_SHARED((2, K), i32),
            pltpu.VMEM((K,), f32),
            pltpu.VMEM((K,), i32),
            pltpu.VMEM((4, H), f32),
            pltpu.VMEM((H,), f32),
            pltpu.VMEM((64,), f32),
            pltpu.VMEM((64,), f32),
            pltpu.VMEM((64,), f32),
            pltpu.VMEM_SHARED((2 * NS, 64), f32),
            pltpu.VMEM((NS, 64), f32),
            pltpu.VMEM((16,), f32),
            pltpu.VMEM((16,), f32),
            pltpu.SemaphoreType.DMA,
        ],
        compiler_params=pltpu.CompilerParams(needs_layout_passes=False),
    )(_sc_body)
    return body(alpha, hs2d, exc, wsc, bsc16)


def kernel(exc, score, excs, hs, W_ih, W_hh, b_ih, b_hh, W_score, b_score,
           attn_k):
    excs1d = excs.reshape(T * E)
    hs1d = hs.reshape(T * H)
    exc2d = exc.reshape(1, E)
    exc3d = exc.reshape(8, 128)
    sc2d = score.reshape(1, 1)
    hlast2d = hs1d[(T - 1) * H:].reshape(1, H)
    wih3 = W_ih.reshape(3, H, 2 * E)
    whh3 = W_hh.reshape(3, H, H)
    bih3 = b_ih.reshape(3, 1, H)
    bhh3 = b_hh.reshape(3, 1, H)
    wsc = W_score.reshape(2 * E)
    bsc16 = jnp.zeros((16,), jnp.float32).at[0].set(b_score[0])

    alpha = _compute_alpha(excs1d, exc3d)
    pred16 = _compute_attention(alpha, hs1d, exc, wsc, bsc16)
    h_new = _compute_gru(wih3, whh3, bih3, bhh3, exc2d, sc2d, hlast2d)

    pred = pred16[0:1].reshape(1, 1)
    return (pred, h_new.reshape(1, 1, H))
